# baseline (speedup 1.0000x reference)
import functools

import jax
import jax.numpy as jnp
from jax import lax
from jax.experimental import pallas as pl
from jax.experimental.pallas import tpu as pltpu
from jax.experimental.pallas import tpu_sc as plsc

TILE = 256
TILE_SH = 512
K = 2
NW = 32
LANES = 16


def _router_body(T, E, NRT, x_ref, wg_ref, dst_ref, wts_ref, eot_ref):
    f32 = jnp.float32
    x = x_ref[...]
    wg = wg_ref[...]
    scores = jax.nn.sigmoid(
        lax.dot_general(x, wg, (((1,), (1,)), ((), ())),
                        preferred_element_type=f32))

    U8 = (lax.broadcasted_iota(jnp.int32, (E, E), 0)
          <= lax.broadcasted_iota(jnp.int32, (E, E), 1)).astype(f32)

    def first_max_onehot(s):
        m = jnp.max(s, axis=1, keepdims=True)
        eq = (s == m).astype(f32)
        c = lax.dot_general(eq, U8, (((1,), (0,)), ((), ())),
                            preferred_element_type=f32)
        return eq * (c == 1.0).astype(f32), m

    oh1, m1 = first_max_onehot(scores)
    oh2, m2 = first_max_onehot(scores - oh1 * 1e9)
    ssum = m1 + m2
    wts_ref[...] = jnp.concatenate([m1 / ssum, m2 / ssum], axis=1)

    mask = oh1 + oh2

    Lt = (lax.broadcasted_iota(jnp.int32, (T, T), 0)
          > lax.broadcasted_iota(jnp.int32, (T, T), 1)).astype(jnp.bfloat16)
    pos = lax.dot_general(Lt, mask.astype(jnp.bfloat16),
                          (((1,), (0,)), ((), ())),
                          preferred_element_type=f32)

    ones_row = jnp.ones((1, T), f32)
    counts = lax.dot_general(ones_row, mask, (((1,), (0,)), ((), ())),
                             preferred_element_type=f32)
    cnt_pad = jnp.ceil(counts / TILE) * TILE
    Us = (lax.broadcasted_iota(jnp.int32, (E, E), 0)
          < lax.broadcasted_iota(jnp.int32, (E, E), 1)).astype(f32)
    off = lax.dot_general(cnt_pad, Us, (((1,), (0,)), ((), ())),
                          preferred_element_type=f32)
    offpos = pos + off
    d1 = jnp.sum(oh1 * offpos, axis=1, keepdims=True)
    d2 = jnp.sum(oh2 * offpos, axis=1, keepdims=True)
    dst_ref[...] = jnp.concatenate([d1, d2], axis=1).astype(jnp.int32)

    ones_col = jnp.ones((T, 1), f32)
    countsc = lax.dot_general(mask, ones_col, (((0,), (0,)), ((), ())),
                              preferred_element_type=f32)
    cnt_padc = jnp.ceil(countsc / TILE) * TILE
    Ls8 = (lax.broadcasted_iota(jnp.int32, (E, E), 0)
           > lax.broadcasted_iota(jnp.int32, (E, E), 1)).astype(f32)
    offc = lax.dot_general(Ls8, cnt_padc, (((1,), (0,)), ((), ())),
                           preferred_element_type=f32)
    endc = offc + cnt_padc
    tstart = lax.broadcasted_iota(jnp.int32, (E, NRT), 1).astype(f32) * TILE
    cmp = (endc <= tstart).astype(f32)
    raw = lax.dot_general(jnp.ones((1, E), f32), cmp, (((1,), (0,)), ((), ())),
                          preferred_element_type=f32)
    widx = jnp.minimum(raw, float(E - 1))
    live = (raw <= float(E - 1)).astype(f32)
    eot_ref[...] = jnp.concatenate([widx, live], axis=0).astype(jnp.int32)


def _swiglu(xb, wg, wu, wd):
    g = lax.dot_general(xb, wg, (((1,), (1,)), ((), ())),
                        preferred_element_type=jnp.float32)
    u = lax.dot_general(xb, wu, (((1,), (1,)), ((), ())),
                        preferred_element_type=jnp.float32)
    h = (g * jax.nn.sigmoid(g) * u).astype(jnp.bfloat16)
    return lax.dot_general(h, wd, (((1,), (1,)), ((), ())),
                           preferred_element_type=jnp.float32)


def _pack_halves(y):
    dh = y.shape[1] // 2
    a = lax.bitcast_convert_type(y[:, :dh].astype(jnp.bfloat16),
                                 jnp.uint16).astype(jnp.uint32)
    b = lax.bitcast_convert_type(y[:, dh:].astype(jnp.bfloat16),
                                 jnp.uint16).astype(jnp.uint32)
    return (a | (b << 16)).astype(jnp.int32)


def _shared_ffn_body(x_ref, wsg_ref, wsu_ref, wsd_ref, out_ref,
                     wgb, wub, wdb):
    @pl.when(pl.program_id(0) == 0)
    def _():
        wgb[...] = wsg_ref[...].astype(jnp.bfloat16)
        wub[...] = wsu_ref[...].astype(jnp.bfloat16)
        wdb[...] = wsd_ref[...].astype(jnp.bfloat16)

    out_ref[...] = _pack_halves(_swiglu(x_ref[...].astype(jnp.bfloat16),
                                        wgb[...], wub[...], wdb[...]))


def _routed_ffn_body(eot_ref, xb_ref, wg_ref, wu_ref, wd_ref, out_ref):
    j = pl.program_id(0)

    @pl.when(eot_ref[1, j] == 1)
    def _():
        out_ref[...] = _pack_halves(_swiglu(xb_ref[...].astype(jnp.bfloat16),
                                            wg_ref[0], wu_ref[0], wd_ref[0]))


def _worker_id():
    return lax.axis_index("s") * 2 + lax.axis_index("c")


def _dispatch_body(T, CP, x_hbm, dst_hbm, xbuf_hbm,
                   idx0, idx1, tok0, tok1, rows0, rows1,
                   gsem0, gsem1, ssem0, ssem1):
    w = _worker_id()
    npairs = K * T // NW
    nc = npairs // CP
    idx = [idx0, idx1]
    tok = [tok0, tok1]
    rows = [rows0, rows1]
    gsem = [gsem0, gsem1]
    ssem = [ssem0, ssem1]

    def start_gather(c):
        b = c % 2
        base = w * npairs + c * CP
        pltpu.sync_copy(dst_hbm.at[pl.ds(base, CP)], idx[b])
        for jj in range(CP // LANES):
            tok[b][pl.ds(jj * LANES, LANES)] = (
                (lax.iota(jnp.int32, LANES) + (base + jj * LANES)) >> 1)
        return pltpu.async_copy(x_hbm.at[tok[b]], rows[b], gsem[b])

    g = start_gather(0)
    scats = [None] * nc
    for c in range(nc):
        g_cur = g
        if c + 1 < nc:
            if c - 1 >= 0:
                scats[c - 1].wait()
            g = start_gather(c + 1)
        g_cur.wait()
        scats[c] = pltpu.async_copy(rows[c % 2], xbuf_hbm.at[idx[c % 2]],
                                    ssem[c % 2])
    scats[nc - 2].wait()
    scats[nc - 1].wait()


def _bcast_lane(vec, lane):
    idx = jnp.full((LANES,), lane, jnp.int32)
    return vec.at[idx].get(mode="promise_in_bounds")


def _combine_body(T, D, CT, obuf_hbm, osh_hbm, dst_hbm, wts_hbm, out_hbm,
                  idx_all, w_all, rows0, rows1, sh0, sh1, out_v,
                  gsem0, gsem1, hsem0, hsem1, osem):
    w = _worker_id()
    TPW = T // NW
    nc = TPW // CT
    rows = [rows0, rows1]
    sh = [sh0, sh1]
    gsem = [gsem0, gsem1]
    hsem = [hsem0, hsem1]
    wbase = w * TPW
    pltpu.sync_copy(dst_hbm.at[pl.ds(2 * wbase, 2 * TPW)], idx_all)
    pltpu.sync_copy(wts_hbm.at[pl.ds(2 * wbase, 2 * TPW)], w_all)

    def start_chunk(c):
        b = c % 2
        g = pltpu.async_copy(obuf_hbm.at[idx_all.at[pl.ds(2 * CT * c, 2 * CT)]],
                             rows[b], gsem[b])
        h = pltpu.async_copy(osh_hbm.at[pl.ds(wbase + c * CT, CT)], sh[b],
                             hsem[b])
        return g, h

    pend = start_chunk(0)
    owait = None
    for c in range(nc):
        b = c % 2
        cur = pend
        if c + 1 < nc:
            pend = start_chunk(c + 1)
        cur[0].wait()
        cur[1].wait()
        w1s, w2s = [], []
        for i in range(CT):
            la, lb = 2 * i, 2 * i + 1
            wc1 = w_all[pl.ds(2 * CT * c + LANES * (la // LANES), LANES)]
            wc2 = w_all[pl.ds(2 * CT * c + LANES * (lb // LANES), LANES)]
            w1s.append(_bcast_lane(wc1, la % LANES))
            w2s.append(_bcast_lane(wc2, lb % LANES))

        if owait is not None:
            owait.wait()

        DH = D // 2

        def unpk(v):
            lo = lax.bitcast_convert_type(v << 16, jnp.float32)
            hi = lax.bitcast_convert_type(v & jnp.int32(-65536), jnp.float32)
            return lo, hi

        def dchunk(k, carry):
            o = k * LANES
            for i in range(CT):
                alo, ahi = unpk(rows[b][2 * i, pl.ds(o, LANES)])
                blo, bhi = unpk(rows[b][2 * i + 1, pl.ds(o, LANES)])
                slo, shi = unpk(sh[b][i, pl.ds(o, LANES)])
                out_v[i, pl.ds(o, LANES)] = (
                    slo + alo * w1s[i] + blo * w2s[i])
                out_v[i, pl.ds(DH + o, LANES)] = (
                    shi + ahi * w1s[i] + bhi * w2s[i])
            return carry

        lax.fori_loop(0, DH // LANES, dchunk, 0)
        owait = pltpu.async_copy(out_v, out_hbm.at[pl.ds(wbase + c * CT, CT)],
                                 osem)
    owait.wait()


def kernel(hidden_states, Wgate, Weg, Weu, Wed, Wsg, Wsu, Wsd):
    orig_shape = hidden_states.shape
    x = hidden_states.reshape(-1, orig_shape[-1])
    T, D = x.shape
    E, FF = Weg.shape[0], Weg.shape[1]
    NP = K * T + E * TILE
    NRT = NP // TILE
    NST = T // TILE_SH

    dst, wts, eot = pl.pallas_call(
        functools.partial(_router_body, T, E, NRT),
        out_shape=(
            jax.ShapeDtypeStruct((T, K), jnp.int32),
            jax.ShapeDtypeStruct((T, K), jnp.float32),
            jax.ShapeDtypeStruct((2, NRT), jnp.int32),
        ),
    )(x, Wgate)
    dstv = dst.reshape(-1)
    wtsv = wts.reshape(-1)

    mesh = plsc.VectorSubcoreMesh(core_axis_name="c", subcore_axis_name="s",
                                  num_cores=2, num_subcores=16)
    CP = 32
    xbuf = pl.kernel(
        functools.partial(_dispatch_body, T, CP),
        out_type=jax.ShapeDtypeStruct((NP, D), jnp.float32),
        mesh=mesh,
        scratch_types=[
            pltpu.VMEM((CP,), jnp.int32),
            pltpu.VMEM((CP,), jnp.int32),
            pltpu.VMEM((CP,), jnp.int32),
            pltpu.VMEM((CP,), jnp.int32),
            pltpu.VMEM((CP, D), jnp.float32),
            pltpu.VMEM((CP, D), jnp.float32),
            pltpu.SemaphoreType.DMA,
            pltpu.SemaphoreType.DMA,
            pltpu.SemaphoreType.DMA,
            pltpu.SemaphoreType.DMA,
        ],
    )(x, dstv)

    osh = pl.pallas_call(
        _shared_ffn_body,
        grid=(NST,),
        in_specs=[
            pl.BlockSpec((TILE_SH, D), lambda j: (j, 0)),
            pl.BlockSpec((FF, D), lambda j: (0, 0)),
            pl.BlockSpec((FF, D), lambda j: (0, 0)),
            pl.BlockSpec((D, FF), lambda j: (0, 0)),
        ],
        out_specs=pl.BlockSpec((TILE_SH, D // 2), lambda j: (j, 0)),
        out_shape=jax.ShapeDtypeStruct((T, D // 2), jnp.int32),
        scratch_shapes=[
            pltpu.VMEM((FF, D), jnp.bfloat16),
            pltpu.VMEM((FF, D), jnp.bfloat16),
            pltpu.VMEM((D, FF), jnp.bfloat16),
        ],
    )(x, Wsg, Wsu, Wsd)

    obuf = pl.pallas_call(
        _routed_ffn_body,
        grid_spec=pltpu.PrefetchScalarGridSpec(
            num_scalar_prefetch=1,
            grid=(NRT,),
            in_specs=[
                pl.BlockSpec((TILE, D), lambda j, eot: (j, 0)),
                pl.BlockSpec((1, FF, D), lambda j, eot: (eot[0, j], 0, 0)),
                pl.BlockSpec((1, FF, D), lambda j, eot: (eot[0, j], 0, 0)),
                pl.BlockSpec((1, D, FF), lambda j, eot: (eot[0, j], 0, 0)),
            ],
            out_specs=pl.BlockSpec((TILE, D // 2), lambda j, eot: (j, 0)),
        ),
        out_shape=jax.ShapeDtypeStruct((NP, D // 2), jnp.int32),
    )(eot, xbuf, Weg.astype(jnp.bfloat16), Weu.astype(jnp.bfloat16),
      Wed.astype(jnp.bfloat16))

    CT = 16
    TPW = T // NW
    out = pl.kernel(
        functools.partial(_combine_body, T, D, CT),
        out_type=jax.ShapeDtypeStruct((T, D), jnp.float32),
        mesh=mesh,
        scratch_types=[
            pltpu.VMEM((2 * TPW,), jnp.int32),
            pltpu.VMEM((2 * TPW,), jnp.float32),
            pltpu.VMEM((2 * CT, D // 2), jnp.int32),
            pltpu.VMEM((2 * CT, D // 2), jnp.int32),
            pltpu.VMEM((CT, D // 2), jnp.int32),
            pltpu.VMEM((CT, D // 2), jnp.int32),
            pltpu.VMEM((CT, D), jnp.float32),
            pltpu.SemaphoreType.DMA,
            pltpu.SemaphoreType.DMA,
            pltpu.SemaphoreType.DMA,
            pltpu.SemaphoreType.DMA,
            pltpu.SemaphoreType.DMA,
        ],
    )(obuf, osh, dstv, wtsv)

    return out.reshape(orig_shape)


# speedup vs baseline: 1.1469x; 1.1469x over previous
import functools

import jax
import jax.numpy as jnp
from jax import lax
from jax.experimental import pallas as pl
from jax.experimental.pallas import tpu as pltpu
from jax.experimental.pallas import tpu_sc as plsc

TILE = 256
TILE_SH = 512
K = 2
NW = 32
LANES = 16


def _router_body(T, E, NRT, x_ref, wg_ref, dst_ref, wts_ref, eot_ref):
    f32 = jnp.float32
    x = x_ref[...]
    wg = wg_ref[...]
    scores = jax.nn.sigmoid(
        lax.dot_general(x, wg, (((1,), (1,)), ((), ())),
                        preferred_element_type=f32))

    U8 = (lax.broadcasted_iota(jnp.int32, (E, E), 0)
          <= lax.broadcasted_iota(jnp.int32, (E, E), 1)).astype(f32)

    def first_max_onehot(s):
        m = jnp.max(s, axis=1, keepdims=True)
        eq = (s == m).astype(f32)
        c = lax.dot_general(eq, U8, (((1,), (0,)), ((), ())),
                            preferred_element_type=f32)
        return eq * (c == 1.0).astype(f32), m

    oh1, m1 = first_max_onehot(scores)
    oh2, m2 = first_max_onehot(scores - oh1 * 1e9)
    ssum = m1 + m2
    wts_ref[...] = jnp.concatenate([m1 / ssum, m2 / ssum], axis=1)

    mask = oh1 + oh2

    Lt = (lax.broadcasted_iota(jnp.int32, (T, T), 0)
          > lax.broadcasted_iota(jnp.int32, (T, T), 1)).astype(jnp.bfloat16)
    pos = lax.dot_general(Lt, mask.astype(jnp.bfloat16),
                          (((1,), (0,)), ((), ())),
                          preferred_element_type=f32)

    ones_row = jnp.ones((1, T), f32)
    counts = lax.dot_general(ones_row, mask, (((1,), (0,)), ((), ())),
                             preferred_element_type=f32)
    cnt_pad = jnp.ceil(counts / TILE) * TILE
    Us = (lax.broadcasted_iota(jnp.int32, (E, E), 0)
          < lax.broadcasted_iota(jnp.int32, (E, E), 1)).astype(f32)
    off = lax.dot_general(cnt_pad, Us, (((1,), (0,)), ((), ())),
                          preferred_element_type=f32)
    offpos = pos + off
    d1 = jnp.sum(oh1 * offpos, axis=1, keepdims=True)
    d2 = jnp.sum(oh2 * offpos, axis=1, keepdims=True)
    dst_ref[...] = jnp.concatenate([d1, d2], axis=1).astype(jnp.int32)

    ones_col = jnp.ones((T, 1), f32)
    countsc = lax.dot_general(mask, ones_col, (((0,), (0,)), ((), ())),
                              preferred_element_type=f32)
    cnt_padc = jnp.ceil(countsc / TILE) * TILE
    Ls8 = (lax.broadcasted_iota(jnp.int32, (E, E), 0)
           > lax.broadcasted_iota(jnp.int32, (E, E), 1)).astype(f32)
    offc = lax.dot_general(Ls8, cnt_padc, (((1,), (0,)), ((), ())),
                           preferred_element_type=f32)
    endc = offc + cnt_padc
    tstart = lax.broadcasted_iota(jnp.int32, (E, NRT), 1).astype(f32) * TILE
    cmp = (endc <= tstart).astype(f32)
    raw = lax.dot_general(jnp.ones((1, E), f32), cmp, (((1,), (0,)), ((), ())),
                          preferred_element_type=f32)
    widx = jnp.minimum(raw, float(E - 1))
    live = (raw <= float(E - 1)).astype(f32)
    eot_ref[...] = jnp.concatenate([widx, live], axis=0).astype(jnp.int32)


def _swiglu(xb, wg, wu, wd):
    g = lax.dot_general(xb, wg, (((1,), (1,)), ((), ())),
                        preferred_element_type=jnp.float32)
    u = lax.dot_general(xb, wu, (((1,), (1,)), ((), ())),
                        preferred_element_type=jnp.float32)
    h = (g * jax.nn.sigmoid(g) * u).astype(jnp.bfloat16)
    return lax.dot_general(h, wd, (((1,), (1,)), ((), ())),
                           preferred_element_type=jnp.float32)


def _pack_halves(y):
    dh = y.shape[1] // 2
    a = lax.bitcast_convert_type(y[:, :dh].astype(jnp.bfloat16),
                                 jnp.uint16).astype(jnp.uint32)
    b = lax.bitcast_convert_type(y[:, dh:].astype(jnp.bfloat16),
                                 jnp.uint16).astype(jnp.uint32)
    return (a | (b << 16)).astype(jnp.int32)


def _shared_ffn_body(x_ref, wsg_ref, wsu_ref, wsd_ref, out_ref,
                     wgb, wub, wdb):
    @pl.when(pl.program_id(0) == 0)
    def _():
        wgb[...] = wsg_ref[...].astype(jnp.bfloat16)
        wub[...] = wsu_ref[...].astype(jnp.bfloat16)
        wdb[...] = wsd_ref[...].astype(jnp.bfloat16)

    out_ref[...] = _pack_halves(_swiglu(x_ref[...].astype(jnp.bfloat16),
                                        wgb[...], wub[...], wdb[...]))


def _routed_ffn_body(eot_ref, xb_ref, wg_ref, wu_ref, wd_ref, out_ref,
                     wgb, wub, wdb):
    j = pl.program_id(0)

    @pl.when(eot_ref[1, j] == 1)
    def _():
        changed = jnp.logical_or(
            j == 0, eot_ref[0, j] != eot_ref[0, jnp.maximum(j - 1, 0)])

        @pl.when(changed)
        def _():
            wgb[...] = wg_ref[0].astype(jnp.bfloat16)
            wub[...] = wu_ref[0].astype(jnp.bfloat16)
            wdb[...] = wd_ref[0].astype(jnp.bfloat16)

        out_ref[...] = _pack_halves(_swiglu(xb_ref[...].astype(jnp.bfloat16),
                                            wgb[...], wub[...], wdb[...]))


def _worker_id():
    return lax.axis_index("s") * 2 + lax.axis_index("c")


def _dispatch_body(T, CP, x_hbm, dst_hbm, xbuf_hbm,
                   idx0, idx1, tok0, tok1, rows0, rows1,
                   gsem0, gsem1, ssem0, ssem1):
    w = _worker_id()
    npairs = K * T // NW
    nc = npairs // CP
    idx = [idx0, idx1]
    tok = [tok0, tok1]
    rows = [rows0, rows1]
    gsem = [gsem0, gsem1]
    ssem = [ssem0, ssem1]

    def start_gather(c):
        b = c % 2
        base = w * npairs + c * CP
        pltpu.sync_copy(dst_hbm.at[pl.ds(base, CP)], idx[b])
        for jj in range(CP // LANES):
            tok[b][pl.ds(jj * LANES, LANES)] = (
                (lax.iota(jnp.int32, LANES) + (base + jj * LANES)) >> 1)
        return pltpu.async_copy(x_hbm.at[tok[b]], rows[b], gsem[b])

    g = start_gather(0)
    scats = [None] * nc
    for c in range(nc):
        g_cur = g
        if c + 1 < nc:
            if c - 1 >= 0:
                scats[c - 1].wait()
            g = start_gather(c + 1)
        g_cur.wait()
        scats[c] = pltpu.async_copy(rows[c % 2], xbuf_hbm.at[idx[c % 2]],
                                    ssem[c % 2])
    scats[nc - 2].wait()
    scats[nc - 1].wait()


def _bcast_lane(vec, lane):
    idx = jnp.full((LANES,), lane, jnp.int32)
    return vec.at[idx].get(mode="promise_in_bounds")


def _combine_body(T, D, CT, obuf_hbm, osh_hbm, dst_hbm, wts_hbm, out_hbm,
                  idx_all, w_all, rows0, rows1, sh0, sh1, out_v,
                  gsem0, gsem1, hsem0, hsem1, osem):
    w = _worker_id()
    TPW = T // NW
    nc = TPW // CT
    rows = [rows0, rows1]
    sh = [sh0, sh1]
    gsem = [gsem0, gsem1]
    hsem = [hsem0, hsem1]
    wbase = w * TPW
    pltpu.sync_copy(dst_hbm.at[pl.ds(2 * wbase, 2 * TPW)], idx_all)
    pltpu.sync_copy(wts_hbm.at[pl.ds(2 * wbase, 2 * TPW)], w_all)

    def start_chunk(c):
        b = c % 2
        g = pltpu.async_copy(obuf_hbm.at[idx_all.at[pl.ds(2 * CT * c, 2 * CT)]],
                             rows[b], gsem[b])
        h = pltpu.async_copy(osh_hbm.at[pl.ds(wbase + c * CT, CT)], sh[b],
                             hsem[b])
        return g, h

    pend = start_chunk(0)
    owait = None
    for c in range(nc):
        b = c % 2
        cur = pend
        if c + 1 < nc:
            pend = start_chunk(c + 1)
        cur[0].wait()
        cur[1].wait()
        w1s, w2s = [], []
        for i in range(CT):
            la, lb = 2 * i, 2 * i + 1
            wc1 = w_all[pl.ds(2 * CT * c + LANES * (la // LANES), LANES)]
            wc2 = w_all[pl.ds(2 * CT * c + LANES * (lb // LANES), LANES)]
            w1s.append(_bcast_lane(wc1, la % LANES))
            w2s.append(_bcast_lane(wc2, lb % LANES))

        if owait is not None:
            owait.wait()

        DH = D // 2

        def unpk(v):
            lo = lax.bitcast_convert_type(v << 16, jnp.float32)
            hi = lax.bitcast_convert_type(v & jnp.int32(-65536), jnp.float32)
            return lo, hi

        def dchunk(k, carry):
            o = k * LANES
            for i in range(CT):
                alo, ahi = unpk(rows[b][2 * i, pl.ds(o, LANES)])
                blo, bhi = unpk(rows[b][2 * i + 1, pl.ds(o, LANES)])
                slo, shi = unpk(sh[b][i, pl.ds(o, LANES)])
                out_v[i, pl.ds(o, LANES)] = (
                    slo + alo * w1s[i] + blo * w2s[i])
                out_v[i, pl.ds(DH + o, LANES)] = (
                    shi + ahi * w1s[i] + bhi * w2s[i])
            return carry

        lax.fori_loop(0, DH // LANES, dchunk, 0)
        owait = pltpu.async_copy(out_v, out_hbm.at[pl.ds(wbase + c * CT, CT)],
                                 osem)
    owait.wait()


def kernel(hidden_states, Wgate, Weg, Weu, Wed, Wsg, Wsu, Wsd):
    orig_shape = hidden_states.shape
    x = hidden_states.reshape(-1, orig_shape[-1])
    T, D = x.shape
    E, FF = Weg.shape[0], Weg.shape[1]
    NP = K * T + E * TILE
    NRT = NP // TILE
    NST = T // TILE_SH

    dst, wts, eot = pl.pallas_call(
        functools.partial(_router_body, T, E, NRT),
        out_shape=(
            jax.ShapeDtypeStruct((T, K), jnp.int32),
            jax.ShapeDtypeStruct((T, K), jnp.float32),
            jax.ShapeDtypeStruct((2, NRT), jnp.int32),
        ),
    )(x, Wgate)
    dstv = dst.reshape(-1)
    wtsv = wts.reshape(-1)

    mesh = plsc.VectorSubcoreMesh(core_axis_name="c", subcore_axis_name="s",
                                  num_cores=2, num_subcores=16)
    CP = 32
    xbuf = pl.kernel(
        functools.partial(_dispatch_body, T, CP),
        out_type=jax.ShapeDtypeStruct((NP, D), jnp.float32),
        mesh=mesh,
        scratch_types=[
            pltpu.VMEM((CP,), jnp.int32),
            pltpu.VMEM((CP,), jnp.int32),
            pltpu.VMEM((CP,), jnp.int32),
            pltpu.VMEM((CP,), jnp.int32),
            pltpu.VMEM((CP, D), jnp.float32),
            pltpu.VMEM((CP, D), jnp.float32),
            pltpu.SemaphoreType.DMA,
            pltpu.SemaphoreType.DMA,
            pltpu.SemaphoreType.DMA,
            pltpu.SemaphoreType.DMA,
        ],
    )(x, dstv)

    osh = pl.pallas_call(
        _shared_ffn_body,
        grid=(NST,),
        in_specs=[
            pl.BlockSpec((TILE_SH, D), lambda j: (j, 0)),
            pl.BlockSpec((FF, D), lambda j: (0, 0)),
            pl.BlockSpec((FF, D), lambda j: (0, 0)),
            pl.BlockSpec((D, FF), lambda j: (0, 0)),
        ],
        out_specs=pl.BlockSpec((TILE_SH, D // 2), lambda j: (j, 0)),
        out_shape=jax.ShapeDtypeStruct((T, D // 2), jnp.int32),
        scratch_shapes=[
            pltpu.VMEM((FF, D), jnp.bfloat16),
            pltpu.VMEM((FF, D), jnp.bfloat16),
            pltpu.VMEM((D, FF), jnp.bfloat16),
        ],
    )(x, Wsg, Wsu, Wsd)

    obuf = pl.pallas_call(
        _routed_ffn_body,
        grid_spec=pltpu.PrefetchScalarGridSpec(
            num_scalar_prefetch=1,
            grid=(NRT,),
            in_specs=[
                pl.BlockSpec((TILE, D), lambda j, eot: (j, 0)),
                pl.BlockSpec((1, FF, D), lambda j, eot: (eot[0, j], 0, 0)),
                pl.BlockSpec((1, FF, D), lambda j, eot: (eot[0, j], 0, 0)),
                pl.BlockSpec((1, D, FF), lambda j, eot: (eot[0, j], 0, 0)),
            ],
            out_specs=pl.BlockSpec((TILE, D // 2), lambda j, eot: (j, 0)),
            scratch_shapes=[
                pltpu.VMEM((FF, D), jnp.bfloat16),
                pltpu.VMEM((FF, D), jnp.bfloat16),
                pltpu.VMEM((D, FF), jnp.bfloat16),
            ],
        ),
        out_shape=jax.ShapeDtypeStruct((NP, D // 2), jnp.int32),
    )(eot, xbuf, Weg, Weu, Wed)

    CT = 16
    TPW = T // NW
    out = pl.kernel(
        functools.partial(_combine_body, T, D, CT),
        out_type=jax.ShapeDtypeStruct((T, D), jnp.float32),
        mesh=mesh,
        scratch_types=[
            pltpu.VMEM((2 * TPW,), jnp.int32),
            pltpu.VMEM((2 * TPW,), jnp.float32),
            pltpu.VMEM((2 * CT, D // 2), jnp.int32),
            pltpu.VMEM((2 * CT, D // 2), jnp.int32),
            pltpu.VMEM((CT, D // 2), jnp.int32),
            pltpu.VMEM((CT, D // 2), jnp.int32),
            pltpu.VMEM((CT, D), jnp.float32),
            pltpu.SemaphoreType.DMA,
            pltpu.SemaphoreType.DMA,
            pltpu.SemaphoreType.DMA,
            pltpu.SemaphoreType.DMA,
            pltpu.SemaphoreType.DMA,
        ],
    )(obuf, osh, dstv, wtsv)

    return out.reshape(orig_shape)
